# baseline (speedup 1.0000x reference)
import functools
import jax
import jax.numpy as jnp
from jax import lax
from jax.experimental import pallas as pl
from jax.experimental.pallas import tpu as pltpu
from jax.experimental.pallas import tpu_sc as plsc

_NC = 2
_NS = 16
_NW = _NC * _NS
_B = 4
_K = 4096
_N = 2048
_WPB = _NW // _B
_CW = _N // _WPB
_RC = 128
_NCHUNK = _K // _RC


def _sc_argmin(x_hbm, out_hbm, buf, amin, aidx, sem0, sem1):
    c = lax.axis_index("c")
    s = lax.axis_index("s")
    wid = s * _NC + c
    b = wid // _WPB
    col0 = (wid % _WPB) * _CW

    for g in range(_CW // 16):
        amin[pl.ds(g * 16, 16)] = jnp.full((16,), jnp.inf, jnp.float32)
        aidx[pl.ds(g * 16, 16)] = jnp.zeros((16,), jnp.int32)

    def chunk_src(i):
        return x_hbm.at[b, pl.ds(i * _RC, _RC), pl.ds(col0, _CW)]

    pltpu.async_copy(chunk_src(0), buf.at[0], sem0)
    pltpu.async_copy(chunk_src(1), buf.at[1], sem1)

    def process(i, slot, sem):
        pltpu.make_async_copy(chunk_src(i), buf.at[slot], sem).wait()
        base = i * _RC
        for g in range(_CW // 16):
            rm0 = amin[pl.ds(g * 16, 16)]
            ri0 = aidx[pl.ds(g * 16, 16)]
            vr0 = jnp.full((16,), 1, jnp.int32) * base

            def row(r, carry):
                rm, ri, vr = carry
                v = buf[slot, r, pl.ds(g * 16, 16)]
                m = v < rm
                return jnp.where(m, v, rm), jnp.where(m, vr, ri), vr + 1

            rm, ri, _ = lax.fori_loop(0, _RC, row, (rm0, ri0, vr0))
            amin[pl.ds(g * 16, 16)] = rm
            aidx[pl.ds(g * 16, 16)] = ri

        nxt = i + 2

        @pl.when(nxt < _NCHUNK)
        def _():
            pltpu.async_copy(chunk_src(nxt), buf.at[slot], sem)

    def loop_body(j, carry):
        process(2 * j, 0, sem0)
        process(2 * j + 1, 1, sem1)
        return carry

    lax.fori_loop(0, _NCHUNK // 2, loop_body, 0)
    pltpu.sync_copy(aidx, out_hbm.at[b, pl.ds(col0, _CW)])


_sc_call = functools.partial(
    pl.kernel,
    out_type=jax.ShapeDtypeStruct((_B, _N), jnp.int32),
    mesh=plsc.VectorSubcoreMesh(core_axis_name="c", subcore_axis_name="s"),
    scratch_types=[
        pltpu.VMEM((2, _RC, _CW), jnp.float32),
        pltpu.VMEM((_CW,), jnp.float32),
        pltpu.VMEM((_CW,), jnp.int32),
        pltpu.SemaphoreType.DMA,
        pltpu.SemaphoreType.DMA,
    ],
)(_sc_argmin)


def kernel(x):
    out = _sc_call(x)
    return out.astype(jnp.int64)


# speedup vs baseline: 1.3404x; 1.3404x over previous
import functools
import jax
import jax.numpy as jnp
from jax import lax
from jax.experimental import pallas as pl
from jax.experimental.pallas import tpu as pltpu
from jax.experimental.pallas import tpu_sc as plsc

_NC = 2
_NS = 16
_NW = _NC * _NS
_B = 4
_K = 4096
_N = 2048
_WPB = _NW // _B
_CW = _N // _WPB
_RC = 128
_NCHUNK = _K // _RC


def _sc_argmin(x_hbm, out_hbm, buf, amin, aidx, sem0, sem1):
    c = lax.axis_index("c")
    s = lax.axis_index("s")
    wid = s * _NC + c
    b = wid // _WPB
    col0 = (wid % _WPB) * _CW

    for g in range(_CW // 16):
        amin[pl.ds(g * 16, 16)] = jnp.full((16,), jnp.inf, jnp.float32)
        aidx[pl.ds(g * 16, 16)] = jnp.zeros((16,), jnp.int32)

    def chunk_src(i):
        return x_hbm.at[b, pl.ds(i * _RC, _RC), pl.ds(col0, _CW)]

    pltpu.async_copy(chunk_src(0), buf.at[0], sem0)
    pltpu.async_copy(chunk_src(1), buf.at[1], sem1)

    def process(i, slot, sem):
        pltpu.make_async_copy(chunk_src(i), buf.at[slot], sem).wait()
        base = i * _RC

        def cg_body(g, carry):
            c16 = g * 16
            rm = amin[pl.ds(c16, 16)]
            ri = aidx[pl.ds(c16, 16)]
            for r in range(_RC):
                v = buf[slot, r, pl.ds(c16, 16)]
                vr = jnp.full((16,), base + r, jnp.int32)
                m = v < rm
                rm = jnp.where(m, v, rm)
                ri = jnp.where(m, vr, ri)
            amin[pl.ds(c16, 16)] = rm
            aidx[pl.ds(c16, 16)] = ri
            return carry

        lax.fori_loop(0, _CW // 16, cg_body, 0)

        nxt = i + 2

        @pl.when(nxt < _NCHUNK)
        def _():
            pltpu.async_copy(chunk_src(nxt), buf.at[slot], sem)

    def loop_body(j, carry):
        process(2 * j, 0, sem0)
        process(2 * j + 1, 1, sem1)
        return carry

    lax.fori_loop(0, _NCHUNK // 2, loop_body, 0)
    pltpu.sync_copy(aidx, out_hbm.at[b, pl.ds(col0, _CW)])


_sc_call = functools.partial(
    pl.kernel,
    out_type=jax.ShapeDtypeStruct((_B, _N), jnp.int32),
    mesh=plsc.VectorSubcoreMesh(core_axis_name="c", subcore_axis_name="s"),
    scratch_types=[
        pltpu.VMEM((2, _RC, _CW), jnp.float32),
        pltpu.VMEM((_CW,), jnp.float32),
        pltpu.VMEM((_CW,), jnp.int32),
        pltpu.SemaphoreType.DMA,
        pltpu.SemaphoreType.DMA,
    ],
)(_sc_argmin)


def kernel(x):
    out = _sc_call(x)
    return out.astype(jnp.int64)


# speedup vs baseline: 2.6082x; 1.9458x over previous
import functools
import jax
import jax.numpy as jnp
from jax import lax
from jax.experimental import pallas as pl
from jax.experimental.pallas import tpu as pltpu
from jax.experimental.pallas import tpu_sc as plsc

_NC = 2
_NS = 16
_NW = _NC * _NS
_B = 4
_K = 4096
_N = 2048
_WPB = _NW // _B
_CW = _N // _WPB
_RC = 128
_NCHUNK = _K // _RC


def _sc_argmin(x_hbm, out_hbm, buf, amin, aidx, sem0, sem1):
    c = lax.axis_index("c")
    s = lax.axis_index("s")
    wid = s * _NC + c
    b = wid // _WPB
    col0 = (wid % _WPB) * _CW

    for g in range(_CW // 16):
        amin[pl.ds(g * 16, 16)] = jnp.full((16,), jnp.inf, jnp.float32)
        aidx[pl.ds(g * 16, 16)] = jnp.zeros((16,), jnp.int32)

    def chunk_src(i):
        return x_hbm.at[b, pl.ds(i * _RC, _RC), pl.ds(col0, _CW)]

    pltpu.async_copy(chunk_src(0), buf.at[0], sem0)
    pltpu.async_copy(chunk_src(1), buf.at[1], sem1)

    def process(i, slot, sem):
        pltpu.make_async_copy(chunk_src(i), buf.at[slot], sem).wait()
        base = i * _RC

        nu = 4

        def cg_body(g, carry):
            c0 = g * (16 * nu)
            rm = [amin[pl.ds(c0 + u * 16, 16)] for u in range(nu)]
            ri = [aidx[pl.ds(c0 + u * 16, 16)] for u in range(nu)]
            for r in range(_RC):
                vr = jnp.full((16,), base + r, jnp.int32)
                for u in range(nu):
                    v = buf[slot, r, pl.ds(c0 + u * 16, 16)]
                    m = v < rm[u]
                    rm[u] = jnp.where(m, v, rm[u])
                    ri[u] = jnp.where(m, vr, ri[u])
            for u in range(nu):
                amin[pl.ds(c0 + u * 16, 16)] = rm[u]
                aidx[pl.ds(c0 + u * 16, 16)] = ri[u]
            return carry

        lax.fori_loop(0, _CW // (16 * nu), cg_body, 0)

        nxt = i + 2

        @pl.when(nxt < _NCHUNK)
        def _():
            pltpu.async_copy(chunk_src(nxt), buf.at[slot], sem)

    def loop_body(j, carry):
        process(2 * j, 0, sem0)
        process(2 * j + 1, 1, sem1)
        return carry

    lax.fori_loop(0, _NCHUNK // 2, loop_body, 0)
    pltpu.sync_copy(aidx, out_hbm.at[b, pl.ds(col0, _CW)])


_sc_call = functools.partial(
    pl.kernel,
    out_type=jax.ShapeDtypeStruct((_B, _N), jnp.int32),
    mesh=plsc.VectorSubcoreMesh(core_axis_name="c", subcore_axis_name="s"),
    scratch_types=[
        pltpu.VMEM((2, _RC, _CW), jnp.float32),
        pltpu.VMEM((_CW,), jnp.float32),
        pltpu.VMEM((_CW,), jnp.int32),
        pltpu.SemaphoreType.DMA,
        pltpu.SemaphoreType.DMA,
    ],
)(_sc_argmin)


def kernel(x):
    out = _sc_call(x)
    return out.astype(jnp.int64)
